# baseline (speedup 1.0000x reference)
import functools

import jax
import jax.numpy as jnp
from jax import lax
from jax.experimental import pallas as pl
from jax.experimental.pallas import tpu as pltpu
from jax.experimental.pallas import tpu_sc as plsc

B = 16384
EMB = 64
FEAT = 32
NF = 26
NW = 32
CHUNK = B // NW
NJ = CHUNK // 128

U_G = 8000512
U_LEN = 7999616
I_G = 800768
I_LEN = 799872
FU_FLD = 32768
NFH = NF // 2


def _untile(xT, nwin, wblk, g0=0, ng=None):
    rows, n = xT.shape
    if ng is None:
        ng = rows // 8
    nin = wblk * 128

    def body(x_ref, o_ref):
        x = x_ref[...]
        o_ref[...] = x.reshape(8, wblk, 128).transpose(1, 0, 2)[None]

    return pl.pallas_call(
        body,
        grid=(ng, pl.cdiv(nwin, wblk)),
        in_specs=[pl.BlockSpec((8, nin), lambda g, w: (g + g0, w))],
        out_specs=pl.BlockSpec((1, wblk, 8, 128), lambda g, w: (g, w, 0, 0)),
        out_shape=jax.ShapeDtypeStruct((ng, nwin, 8, 128), jnp.float32),
    )(xT)


def _ui_body(uid_hbm, iid_hbm, u_hbm, i_hbm, ub_hbm, ib_hbm, out_hbm,
             uid_v, iid_v, jji_v, ub_v, ib_v, ubuf, ibuf, out_v, sem):
    wid = lax.axis_index("s") * 2 + lax.axis_index("c")

    jrow = wid * NJ
    pltpu.sync_copy(uid_hbm.at[pl.ds(jrow, NJ), :], uid_v)
    pltpu.sync_copy(iid_hbm.at[pl.ds(jrow, NJ), :], iid_v)

    bias = []
    for j in range(NJ):
        dst = pl.ds(j * 128, 128)
        bias.append(pltpu.async_copy(ub_hbm.at[uid_v.at[j]],
                                     ub_v.at[dst], sem))
        bias.append(pltpu.async_copy(ib_hbm.at[iid_v.at[j]],
                                     ib_v.at[dst], sem))
    for c in bias:
        c.wait()

    def mkjj(g, _):
        j = g // 8
        sl = pl.ds((g % 8) * 16, 16)
        u16 = uid_v[j, sl]
        uid_v[j, sl] = ((u16 >> 7) << 10) + (u16 & 127)
        i16 = iid_v[j, sl]
        jji_v[j, sl] = ((i16 >> 7) << 10) + (i16 & 127)
        return 0

    lax.fori_loop(0, NJ * 8, mkjj, 0)

    def drain(n):
        for _ in range(n):
            pltpu.make_async_copy(ub_hbm.at[iid_v.at[0]],
                                  ub_v.at[pl.ds(0, 128)], sem).wait()

    for r in range(2):
        def ui_fire(d, _):
            ub = (d >> 3) * U_G + (d & 7) * 128
            ib = (d >> 3) * I_G + (d & 7) * 128
            for j2 in range(2):
                jq = r * 2 + j2
                dst = pl.ds(j2 * 128, 128)
                pltpu.async_copy(u_hbm.at[pl.ds(ub, U_LEN)].at[uid_v.at[jq]],
                                 ubuf.at[d, dst], sem)
                pltpu.async_copy(i_hbm.at[pl.ds(ib, I_LEN)].at[jji_v.at[jq]],
                                 ibuf.at[d, dst], sem)

            @pl.when(d >= 4)
            def _():
                drain(4)
            return 0

        lax.fori_loop(0, EMB, ui_fire, 0)
        drain(16)

        def ui_group(g, _):
            sl = pl.ds(r * 256 + g * 16, 16)
            gsl = pl.ds(g * 16, 16)
            acc = ub_v[sl] + ib_v[sl]
            for d in range(EMB):
                acc = acc + ubuf[d, gsl] * ibuf[d, gsl]
            out_v[sl] = acc
            return 0

        lax.fori_loop(0, 16, ui_group, 0)

    pltpu.sync_copy(out_v, out_hbm.at[pl.ds(wid * CHUNK, CHUNK)])


def _fields_body(f0, nf, iid_hbm, xf_hbm, fi_hbm, fu_hbm, out_hbm,
                 iid_v, jji_v, xf_v, ftab_v, fib_v, out_v, sem, fsem):
    wid = lax.axis_index("s") * 2 + lax.axis_index("c")
    jrow = wid * NJ
    pltpu.sync_copy(iid_hbm.at[pl.ds(jrow, NJ), :], iid_v)

    def mkjj(g, _):
        j = g // 8
        sl = pl.ds((g % 8) * 16, 16)
        i16 = iid_v[j, sl]
        jji_v[j, sl] = ((i16 >> 7) << 10) + (i16 & 127)
        return 0

    lax.fori_loop(0, NJ * 8, mkjj, 0)

    def zero(g, _):
        out_v[pl.ds(g * 16, 16)] = jnp.zeros((16,), jnp.float32)
        return 0

    lax.fori_loop(0, NW, zero, 0)

    def drain(n):
        for _ in range(n):
            pltpu.make_async_copy(fu_hbm.at[pl.ds(0, 128)],
                                  ftab_v.at[pl.ds(0, 128)], sem).wait()

    def fire_field(floc, p):
        def fd(d, _):
            base = (floc * 4 + (d >> 3)) * I_G + (d & 7) * 128
            for j in range(NJ):
                pltpu.async_copy(
                    fi_hbm.at[pl.ds(base, I_LEN)].at[jji_v.at[j]],
                    fib_v.at[p, d, pl.ds(j * 128, 128)], sem)
            return 0

        lax.fori_loop(0, FEAT, fd, 0)

    fire_field(0, 0)

    def field(floc, _):
        p = lax.rem(floc, 2)
        fg = floc + f0
        ft = pltpu.async_copy(fu_hbm.at[pl.ds(fg * FU_FLD, FU_FLD)], ftab_v,
                              fsem)
        xc = pltpu.async_copy(xf_hbm.at[fg, pl.ds(jrow, NJ), :], xf_v, fsem)

        @pl.when(floc < nf - 1)
        def _():
            fire_field(floc + 1, 1 - p)

        drain(FEAT * NJ)
        ft.wait()
        xc.wait()

        def fgroup(g, _):
            j = g // 8
            sl16 = pl.ds((g % 8) * 16, 16)
            x16 = xf_v[j, sl16]
            xj = ((x16 >> 7) << 10) + (x16 & 127)
            sl = pl.ds(g * 16, 16)
            acc = out_v[sl]
            for d in range(FEAT):
                idx = xj + ((d >> 3) * 8192 + (d & 7) * 128)
                fuv = plsc.load_gather(ftab_v, [idx])
                acc = acc + fuv * fib_v[p, d, sl]
            out_v[sl] = acc
            return 0

        lax.fori_loop(0, NW, fgroup, 0)
        return 0

    lax.fori_loop(0, nf, field, 0)

    pltpu.sync_copy(out_v, out_hbm.at[pl.ds(wid * CHUNK, CHUNK)])


def _sc_ui(mesh, u2, i2, u_t, i_t, ubf, ibf):
    run = pl.kernel(
        _ui_body,
        out_type=jax.ShapeDtypeStruct((B,), jnp.float32),
        mesh=mesh,
        compiler_params=pltpu.CompilerParams(
            needs_layout_passes=False, use_tc_tiling_on_sc=False),
        scratch_types=[
            pltpu.VMEM((NJ, 128), jnp.int32),
            pltpu.VMEM((NJ, 128), jnp.int32),
            pltpu.VMEM((NJ, 128), jnp.int32),
            pltpu.VMEM((CHUNK,), jnp.float32),
            pltpu.VMEM((CHUNK,), jnp.float32),
            pltpu.VMEM((EMB, 256), jnp.float32),
            pltpu.VMEM((EMB, 256), jnp.float32),
            pltpu.VMEM((CHUNK,), jnp.float32),
            pltpu.SemaphoreType.DMA,
        ],
    )
    return run(u2, i2, u_t, i_t, ubf, ibf)


def _sc_fields(mesh, f0, nf, i2, xf3, fi_t, fu_t):
    run = pl.kernel(
        functools.partial(_fields_body, f0, nf),
        out_type=jax.ShapeDtypeStruct((B,), jnp.float32),
        mesh=mesh,
        compiler_params=pltpu.CompilerParams(
            needs_layout_passes=False, use_tc_tiling_on_sc=False),
        scratch_types=[
            pltpu.VMEM((NJ, 128), jnp.int32),
            pltpu.VMEM((NJ, 128), jnp.int32),
            pltpu.VMEM((NJ, 128), jnp.int32),
            pltpu.VMEM((FU_FLD,), jnp.float32),
            pltpu.VMEM((2, FEAT, CHUNK), jnp.float32),
            pltpu.VMEM((CHUNK,), jnp.float32),
            pltpu.SemaphoreType.DMA,
            pltpu.SemaphoreType.DMA,
        ],
    )
    return run(i2, xf3, fi_t, fu_t)


def kernel(u_id, i_id, features, user_emb, user_bias, item_emb, item_bias,
           feat_u, feat_i, mean):
    uT = user_emb.T
    iT = item_emb.T
    fuT = jnp.transpose(feat_u, (0, 2, 1)).reshape(NF * FEAT, 1000)
    fiT = jnp.transpose(feat_i, (0, 2, 1)).reshape(NF * FEAT, 100000)

    i_t = _untile(iT, 782, 782).reshape(-1)
    u_t = _untile(uT, 7813, 1024).reshape(-1)
    ubf = user_bias.reshape(-1)
    ibf = item_bias.reshape(-1)

    u2 = u_id.astype(jnp.int32).reshape(B // 128, 128)
    i2 = i_id.astype(jnp.int32).reshape(B // 128, 128)
    xf3 = features.astype(jnp.int32).T.reshape(NF, B // 128, 128)

    mesh = plsc.VectorSubcoreMesh(core_axis_name="c", subcore_axis_name="s")
    out_ui = _sc_ui(mesh, u2, i2, u_t, i_t, ubf, ibf)

    fiT, fuT = lax.optimization_barrier((fiT, fuT, u_t, i_t, ubf, ibf))[:2]
    fu_t = _untile(fuT, 8, 8).reshape(-1)
    fi_a = _untile(fiT, 782, 782, 0, NFH * 4).reshape(-1)
    out_fa = _sc_fields(mesh, 0, NFH, i2, xf3, fi_a, fu_t)
    fiT2 = lax.optimization_barrier((fiT, fi_a))[0]
    fi_b = _untile(fiT2, 782, 782, NFH * 4, (NF - NFH) * 4).reshape(-1)
    out_fb = _sc_fields(mesh, NFH, NF - NFH, i2, xf3, fi_b, fu_t)
    return out_ui + out_fa + out_fb + mean[0]


# speedup vs baseline: 1.0066x; 1.0066x over previous
import functools

import jax
import jax.numpy as jnp
from jax import lax
from jax.experimental import pallas as pl
from jax.experimental.pallas import tpu as pltpu
from jax.experimental.pallas import tpu_sc as plsc

B = 16384
EMB = 64
FEAT = 32
NF = 26
NW = 32
CHUNK = B // NW
NJ = CHUNK // 128

U_G = 8000512
U_LEN = 7999616
I_G = 800768
I_LEN = 799872
FU_FLD = 32768
NFH = NF // 2


def _untile(xT, nwin, wblk, g0=0, ng=None):
    rows, n = xT.shape
    if ng is None:
        ng = rows // 8
    nin = wblk * 128

    def body(x_ref, o_ref):
        x = x_ref[...]
        o_ref[...] = x.reshape(8, wblk, 128).transpose(1, 0, 2)[None]

    return pl.pallas_call(
        body,
        grid=(ng, pl.cdiv(nwin, wblk)),
        in_specs=[pl.BlockSpec((8, nin), lambda g, w: (g + g0, w))],
        out_specs=pl.BlockSpec((1, wblk, 8, 128), lambda g, w: (g, w, 0, 0)),
        out_shape=jax.ShapeDtypeStruct((ng, nwin, 8, 128), jnp.float32),
    )(xT)


def _ui_body(uid_hbm, iid_hbm, u_hbm, i_hbm, ub_hbm, ib_hbm, out_hbm,
             uid_v, iid_v, jji_v, ub_v, ib_v, ubuf, ibuf, out_v, sem):
    wid = lax.axis_index("s") * 2 + lax.axis_index("c")

    jrow = wid * NJ
    pltpu.sync_copy(uid_hbm.at[pl.ds(jrow, NJ), :], uid_v)
    pltpu.sync_copy(iid_hbm.at[pl.ds(jrow, NJ), :], iid_v)

    bias = []
    for j in range(NJ):
        dst = pl.ds(j * 128, 128)
        bias.append(pltpu.async_copy(ub_hbm.at[uid_v.at[j]],
                                     ub_v.at[dst], sem))
        bias.append(pltpu.async_copy(ib_hbm.at[iid_v.at[j]],
                                     ib_v.at[dst], sem))
    for c in bias:
        c.wait()

    def mkjj(g, _):
        j = g // 8
        sl = pl.ds((g % 8) * 16, 16)
        u16 = uid_v[j, sl]
        uid_v[j, sl] = ((u16 >> 7) << 10) + (u16 & 127)
        i16 = iid_v[j, sl]
        jji_v[j, sl] = ((i16 >> 7) << 10) + (i16 & 127)
        return 0

    lax.fori_loop(0, NJ * 8, mkjj, 0)

    def drain(n):
        for _ in range(n):
            pltpu.make_async_copy(ub_hbm.at[iid_v.at[0]],
                                  ub_v.at[pl.ds(0, 128)], sem).wait()

    for r in range(2):
        def ui_fire(d, _):
            ub = (d >> 3) * U_G + (d & 7) * 128
            ib = (d >> 3) * I_G + (d & 7) * 128
            for j2 in range(2):
                jq = r * 2 + j2
                dst = pl.ds(j2 * 128, 128)
                pltpu.async_copy(u_hbm.at[pl.ds(ub, U_LEN)].at[uid_v.at[jq]],
                                 ubuf.at[d, dst], sem)
                pltpu.async_copy(i_hbm.at[pl.ds(ib, I_LEN)].at[jji_v.at[jq]],
                                 ibuf.at[d, dst], sem)

            @pl.when(d >= 4)
            def _():
                drain(4)
            return 0

        lax.fori_loop(0, EMB, ui_fire, 0)
        drain(16)

        def ui_group(g, _):
            sl = pl.ds(r * 256 + g * 16, 16)
            gsl = pl.ds(g * 16, 16)
            acc = ub_v[sl] + ib_v[sl]
            for d in range(EMB):
                acc = acc + ubuf[d, gsl] * ibuf[d, gsl]
            out_v[sl] = acc
            return 0

        lax.fori_loop(0, 16, ui_group, 0)

    pltpu.sync_copy(out_v, out_hbm.at[pl.ds(wid * CHUNK, CHUNK)])


def _fields_body(f0, nf, iid_hbm, xf_hbm, fi_hbm, fu_hbm, out_hbm,
                 iid_v, jji_v, xf_v, ftab_v, fib_v, out_v, sem, fsem):
    wid = lax.axis_index("s") * 2 + lax.axis_index("c")
    jrow = wid * NJ
    pltpu.sync_copy(iid_hbm.at[pl.ds(jrow, NJ), :], iid_v)

    def mkjj(g, _):
        j = g // 8
        sl = pl.ds((g % 8) * 16, 16)
        i16 = iid_v[j, sl]
        jji_v[j, sl] = ((i16 >> 7) << 10) + (i16 & 127)
        return 0

    lax.fori_loop(0, NJ * 8, mkjj, 0)

    def zero(g, _):
        out_v[pl.ds(g * 16, 16)] = jnp.zeros((16,), jnp.float32)
        return 0

    lax.fori_loop(0, NW, zero, 0)

    def drain(n):
        for _ in range(n):
            pltpu.make_async_copy(fu_hbm.at[pl.ds(0, 128)],
                                  ftab_v.at[pl.ds(0, 128)], sem).wait()

    def fire_field(floc, p):
        def fd(d, _):
            base = (floc * 4 + (d >> 3)) * I_G + (d & 7) * 128
            for j in range(NJ):
                pltpu.async_copy(
                    fi_hbm.at[pl.ds(base, I_LEN)].at[jji_v.at[j]],
                    fib_v.at[p, d, pl.ds(j * 128, 128)], sem)
            return 0

        lax.fori_loop(0, FEAT, fd, 0)

    fire_field(0, 0)

    def field(floc, _):
        p = lax.rem(floc, 2)
        fg = floc + f0
        ft = pltpu.async_copy(fu_hbm.at[pl.ds(fg * FU_FLD, FU_FLD)], ftab_v,
                              fsem)
        xc = pltpu.async_copy(xf_hbm.at[fg, pl.ds(jrow, NJ), :], xf_v, fsem)

        @pl.when(floc < nf - 1)
        def _():
            fire_field(floc + 1, 1 - p)

        drain(FEAT * NJ)
        ft.wait()
        xc.wait()

        def fgroup(g, _):
            j = g // 8
            sl16 = pl.ds((g % 8) * 16, 16)
            x16 = xf_v[j, sl16]
            xj = ((x16 >> 7) << 10) + (x16 & 127)
            sl = pl.ds(g * 16, 16)
            acc = out_v[sl]
            for d in range(FEAT):
                idx = xj + ((d >> 3) * 8192 + (d & 7) * 128)
                fuv = plsc.load_gather(ftab_v, [idx])
                acc = acc + fuv * fib_v[p, d, sl]
            out_v[sl] = acc
            return 0

        lax.fori_loop(0, NW, fgroup, 0)
        return 0

    lax.fori_loop(0, nf, field, 0)

    pltpu.sync_copy(out_v, out_hbm.at[pl.ds(wid * CHUNK, CHUNK)])


def _sc_ui(mesh, u2, i2, u_t, i_t, ubf, ibf):
    run = pl.kernel(
        _ui_body,
        out_type=jax.ShapeDtypeStruct((B,), jnp.float32),
        mesh=mesh,
        compiler_params=pltpu.CompilerParams(
            needs_layout_passes=False, use_tc_tiling_on_sc=False),
        scratch_types=[
            pltpu.VMEM((NJ, 128), jnp.int32),
            pltpu.VMEM((NJ, 128), jnp.int32),
            pltpu.VMEM((NJ, 128), jnp.int32),
            pltpu.VMEM((CHUNK,), jnp.float32),
            pltpu.VMEM((CHUNK,), jnp.float32),
            pltpu.VMEM((EMB, 256), jnp.float32),
            pltpu.VMEM((EMB, 256), jnp.float32),
            pltpu.VMEM((CHUNK,), jnp.float32),
            pltpu.SemaphoreType.DMA,
        ],
    )
    return run(u2, i2, u_t, i_t, ubf, ibf)


def _sc_fields(mesh, f0, nf, i2, xf3, fi_t, fu_t):
    run = pl.kernel(
        functools.partial(_fields_body, f0, nf),
        out_type=jax.ShapeDtypeStruct((B,), jnp.float32),
        mesh=mesh,
        compiler_params=pltpu.CompilerParams(
            needs_layout_passes=False, use_tc_tiling_on_sc=False),
        scratch_types=[
            pltpu.VMEM((NJ, 128), jnp.int32),
            pltpu.VMEM((NJ, 128), jnp.int32),
            pltpu.VMEM((NJ, 128), jnp.int32),
            pltpu.VMEM((FU_FLD,), jnp.float32),
            pltpu.VMEM((2, FEAT, CHUNK), jnp.float32),
            pltpu.VMEM((CHUNK,), jnp.float32),
            pltpu.SemaphoreType.DMA,
            pltpu.SemaphoreType.DMA,
        ],
    )
    return run(i2, xf3, fi_t, fu_t)


def kernel(u_id, i_id, features, user_emb, user_bias, item_emb, item_bias,
           feat_u, feat_i, mean):
    uT = user_emb.T
    iT = item_emb.T
    fuT = jnp.transpose(feat_u, (0, 2, 1)).reshape(NF * FEAT, 1000)
    fiT = jnp.transpose(feat_i, (0, 2, 1)).reshape(NF * FEAT, 100000)

    i_t = _untile(iT, 782, 391).reshape(-1)
    u_t = _untile(uT, 7813, 512).reshape(-1)
    ubf = user_bias.reshape(-1)
    ibf = item_bias.reshape(-1)

    u2 = u_id.astype(jnp.int32).reshape(B // 128, 128)
    i2 = i_id.astype(jnp.int32).reshape(B // 128, 128)
    xf3 = features.astype(jnp.int32).T.reshape(NF, B // 128, 128)

    mesh = plsc.VectorSubcoreMesh(core_axis_name="c", subcore_axis_name="s")
    out_ui = _sc_ui(mesh, u2, i2, u_t, i_t, ubf, ibf)

    fiT, fuT = lax.optimization_barrier((fiT, fuT, u_t, i_t, ubf, ibf))[:2]
    fu_t = _untile(fuT, 8, 8).reshape(-1)
    fi_a = _untile(fiT, 782, 391, 0, NFH * 4).reshape(-1)
    out_fa = _sc_fields(mesh, 0, NFH, i2, xf3, fi_a, fu_t)
    fiT2 = lax.optimization_barrier((fiT, fi_a))[0]
    fi_b = _untile(fiT2, 782, 391, NFH * 4, (NF - NFH) * 4).reshape(-1)
    out_fb = _sc_fields(mesh, NFH, NF - NFH, i2, xf3, fi_b, fu_t)
    return out_ui + out_fa + out_fb + mean[0]


# speedup vs baseline: 1.0203x; 1.0136x over previous
import functools

import jax
import jax.numpy as jnp
from jax import lax
from jax.experimental import pallas as pl
from jax.experimental.pallas import tpu as pltpu
from jax.experimental.pallas import tpu_sc as plsc

B = 16384
EMB = 64
FEAT = 32
NF = 26
NW = 32
CHUNK = B // NW
NJ = CHUNK // 128

U_G = 8000512
U_LEN = 7999616
I_G = 800768
I_LEN = 799872
FU_FLD = 32768
NFH = NF // 2


def _untile(xT, nwin, wblk, g0=0, ng=None):
    rows, n = xT.shape
    if ng is None:
        ng = rows // 8
    nin = wblk * 128

    def body(x_ref, o_ref):
        x = x_ref[...]
        o_ref[...] = x.reshape(8, wblk, 128).transpose(1, 0, 2)[None]

    return pl.pallas_call(
        body,
        grid=(ng, pl.cdiv(nwin, wblk)),
        in_specs=[pl.BlockSpec((8, nin), lambda g, w: (g + g0, w))],
        out_specs=pl.BlockSpec((1, wblk, 8, 128), lambda g, w: (g, w, 0, 0)),
        out_shape=jax.ShapeDtypeStruct((ng, nwin, 8, 128), jnp.float32),
    )(xT)


def _ui_body(uid_hbm, iid_hbm, u_hbm, i_hbm, ub_hbm, ib_hbm, out_hbm,
             uid_v, iid_v, jji_v, ub_v, ib_v, ubuf, ibuf, out_v, sem):
    wid = lax.axis_index("s") * 2 + lax.axis_index("c")

    jrow = wid * NJ
    pltpu.sync_copy(uid_hbm.at[pl.ds(jrow, NJ), :], uid_v)
    pltpu.sync_copy(iid_hbm.at[pl.ds(jrow, NJ), :], iid_v)

    bias = []
    for j in range(NJ):
        dst = pl.ds(j * 128, 128)
        bias.append(pltpu.async_copy(ub_hbm.at[uid_v.at[j]],
                                     ub_v.at[dst], sem))
        bias.append(pltpu.async_copy(ib_hbm.at[iid_v.at[j]],
                                     ib_v.at[dst], sem))
    for c in bias:
        c.wait()

    def mkjj(g, _):
        j = g // 8
        sl = pl.ds((g % 8) * 16, 16)
        u16 = uid_v[j, sl]
        uid_v[j, sl] = ((u16 >> 7) << 10) + (u16 & 127)
        i16 = iid_v[j, sl]
        jji_v[j, sl] = ((i16 >> 7) << 10) + (i16 & 127)
        return 0

    lax.fori_loop(0, NJ * 8, mkjj, 0)

    def drain(n):
        for _ in range(n):
            pltpu.make_async_copy(ub_hbm.at[iid_v.at[0]],
                                  ub_v.at[pl.ds(0, 128)], sem).wait()

    for r in range(2):
        def ui_fire(d, _):
            ub = (d >> 3) * U_G + (d & 7) * 128
            ib = (d >> 3) * I_G + (d & 7) * 128
            for j2 in range(2):
                jq = r * 2 + j2
                dst = pl.ds(j2 * 128, 128)
                pltpu.async_copy(u_hbm.at[pl.ds(ub, U_LEN)].at[uid_v.at[jq]],
                                 ubuf.at[d, dst], sem)
                pltpu.async_copy(i_hbm.at[pl.ds(ib, I_LEN)].at[jji_v.at[jq]],
                                 ibuf.at[d, dst], sem)

            @pl.when(d >= 24)
            def _():
                drain(4)
            return 0

        lax.fori_loop(0, EMB, ui_fire, 0)
        drain(96)

        def ui_group(g, _):
            sl = pl.ds(r * 256 + g * 16, 16)
            gsl = pl.ds(g * 16, 16)
            acc = ub_v[sl] + ib_v[sl]
            for d in range(EMB):
                acc = acc + ubuf[d, gsl] * ibuf[d, gsl]
            out_v[sl] = acc
            return 0

        lax.fori_loop(0, 16, ui_group, 0)

    pltpu.sync_copy(out_v, out_hbm.at[pl.ds(wid * CHUNK, CHUNK)])


def _fields_body(f0, nf, iid_hbm, xf_hbm, fi_hbm, fu_hbm, out_hbm,
                 iid_v, jji_v, xf_v, ftab_v, fib_v, out_v, sem, fsem):
    wid = lax.axis_index("s") * 2 + lax.axis_index("c")
    jrow = wid * NJ
    pltpu.sync_copy(iid_hbm.at[pl.ds(jrow, NJ), :], iid_v)

    def mkjj(g, _):
        j = g // 8
        sl = pl.ds((g % 8) * 16, 16)
        i16 = iid_v[j, sl]
        jji_v[j, sl] = ((i16 >> 7) << 10) + (i16 & 127)
        return 0

    lax.fori_loop(0, NJ * 8, mkjj, 0)

    def zero(g, _):
        out_v[pl.ds(g * 16, 16)] = jnp.zeros((16,), jnp.float32)
        return 0

    lax.fori_loop(0, NW, zero, 0)

    def drain(n):
        for _ in range(n):
            pltpu.make_async_copy(fu_hbm.at[pl.ds(0, 128)],
                                  ftab_v.at[pl.ds(0, 128)], sem).wait()

    def fire_field(floc, p):
        def fd(d, _):
            base = (floc * 4 + (d >> 3)) * I_G + (d & 7) * 128
            for j in range(NJ):
                pltpu.async_copy(
                    fi_hbm.at[pl.ds(base, I_LEN)].at[jji_v.at[j]],
                    fib_v.at[p, d, pl.ds(j * 128, 128)], sem)
            return 0

        lax.fori_loop(0, FEAT, fd, 0)

    fire_field(0, 0)

    def field(floc, _):
        p = lax.rem(floc, 2)
        fg = floc + f0
        ft = pltpu.async_copy(fu_hbm.at[pl.ds(fg * FU_FLD, FU_FLD)], ftab_v,
                              fsem)
        xc = pltpu.async_copy(xf_hbm.at[fg, pl.ds(jrow, NJ), :], xf_v, fsem)

        @pl.when(floc < nf - 1)
        def _():
            fire_field(floc + 1, 1 - p)

        drain(FEAT * NJ)
        ft.wait()
        xc.wait()

        def fgroup(g, _):
            j = g // 8
            sl16 = pl.ds((g % 8) * 16, 16)
            x16 = xf_v[j, sl16]
            xj = ((x16 >> 7) << 10) + (x16 & 127)
            sl = pl.ds(g * 16, 16)
            acc = out_v[sl]
            for d in range(FEAT):
                idx = xj + ((d >> 3) * 8192 + (d & 7) * 128)
                fuv = plsc.load_gather(ftab_v, [idx])
                acc = acc + fuv * fib_v[p, d, sl]
            out_v[sl] = acc
            return 0

        lax.fori_loop(0, NW, fgroup, 0)
        return 0

    lax.fori_loop(0, nf, field, 0)

    pltpu.sync_copy(out_v, out_hbm.at[pl.ds(wid * CHUNK, CHUNK)])


def _sc_ui(mesh, u2, i2, u_t, i_t, ubf, ibf):
    run = pl.kernel(
        _ui_body,
        out_type=jax.ShapeDtypeStruct((B,), jnp.float32),
        mesh=mesh,
        compiler_params=pltpu.CompilerParams(
            needs_layout_passes=False, use_tc_tiling_on_sc=False),
        scratch_types=[
            pltpu.VMEM((NJ, 128), jnp.int32),
            pltpu.VMEM((NJ, 128), jnp.int32),
            pltpu.VMEM((NJ, 128), jnp.int32),
            pltpu.VMEM((CHUNK,), jnp.float32),
            pltpu.VMEM((CHUNK,), jnp.float32),
            pltpu.VMEM((EMB, 256), jnp.float32),
            pltpu.VMEM((EMB, 256), jnp.float32),
            pltpu.VMEM((CHUNK,), jnp.float32),
            pltpu.SemaphoreType.DMA,
        ],
    )
    return run(u2, i2, u_t, i_t, ubf, ibf)


def _sc_fields(mesh, f0, nf, i2, xf3, fi_t, fu_t):
    run = pl.kernel(
        functools.partial(_fields_body, f0, nf),
        out_type=jax.ShapeDtypeStruct((B,), jnp.float32),
        mesh=mesh,
        compiler_params=pltpu.CompilerParams(
            needs_layout_passes=False, use_tc_tiling_on_sc=False),
        scratch_types=[
            pltpu.VMEM((NJ, 128), jnp.int32),
            pltpu.VMEM((NJ, 128), jnp.int32),
            pltpu.VMEM((NJ, 128), jnp.int32),
            pltpu.VMEM((FU_FLD,), jnp.float32),
            pltpu.VMEM((2, FEAT, CHUNK), jnp.float32),
            pltpu.VMEM((CHUNK,), jnp.float32),
            pltpu.SemaphoreType.DMA,
            pltpu.SemaphoreType.DMA,
        ],
    )
    return run(i2, xf3, fi_t, fu_t)


def kernel(u_id, i_id, features, user_emb, user_bias, item_emb, item_bias,
           feat_u, feat_i, mean):
    uT = user_emb.T
    iT = item_emb.T
    fuT = jnp.transpose(feat_u, (0, 2, 1)).reshape(NF * FEAT, 1000)
    fiT = jnp.transpose(feat_i, (0, 2, 1)).reshape(NF * FEAT, 100000)

    i_t = _untile(iT, 782, 391).reshape(-1)
    u_t = _untile(uT, 7813, 512).reshape(-1)
    ubf = user_bias.reshape(-1)
    ibf = item_bias.reshape(-1)

    u2 = u_id.astype(jnp.int32).reshape(B // 128, 128)
    i2 = i_id.astype(jnp.int32).reshape(B // 128, 128)
    xf3 = features.astype(jnp.int32).T.reshape(NF, B // 128, 128)

    mesh = plsc.VectorSubcoreMesh(core_axis_name="c", subcore_axis_name="s")
    out_ui = _sc_ui(mesh, u2, i2, u_t, i_t, ubf, ibf)

    fiT, fuT = lax.optimization_barrier((fiT, fuT, u_t, i_t, ubf, ibf))[:2]
    fu_t = _untile(fuT, 8, 8).reshape(-1)
    fi_a = _untile(fiT, 782, 391, 0, NFH * 4).reshape(-1)
    out_fa = _sc_fields(mesh, 0, NFH, i2, xf3, fi_a, fu_t)
    fiT2 = lax.optimization_barrier((fiT, fi_a))[0]
    fi_b = _untile(fiT2, 782, 391, NFH * 4, (NF - NFH) * 4).reshape(-1)
    out_fb = _sc_fields(mesh, NFH, NF - NFH, i2, xf3, fi_b, fu_t)
    return out_ui + out_fa + out_fb + mean[0]


# speedup vs baseline: 1.0260x; 1.0056x over previous
import functools

import jax
import jax.numpy as jnp
from jax import lax
from jax.experimental import pallas as pl
from jax.experimental.pallas import tpu as pltpu
from jax.experimental.pallas import tpu_sc as plsc

B = 16384
EMB = 64
FEAT = 32
NF = 26
NW = 32
CHUNK = B // NW
NJ = CHUNK // 128

U_G = 8000512
U_LEN = 7999616
I_G = 800768
I_LEN = 799872
FU_FLD = 32768
NFH = NF // 2


def _untile(xT, nwin, wblk, g0=0, ng=None):
    rows, n = xT.shape
    if ng is None:
        ng = rows // 8
    nin = wblk * 128

    def body(x_ref, o_ref):
        x = x_ref[...]
        o_ref[...] = x.reshape(8, wblk, 128).transpose(1, 0, 2)[None]

    return pl.pallas_call(
        body,
        grid=(ng, pl.cdiv(nwin, wblk)),
        in_specs=[pl.BlockSpec((8, nin), lambda g, w: (g + g0, w))],
        out_specs=pl.BlockSpec((1, wblk, 8, 128), lambda g, w: (g, w, 0, 0)),
        out_shape=jax.ShapeDtypeStruct((ng, nwin, 8, 128), jnp.float32),
    )(xT)


def _ui_body(uid_hbm, iid_hbm, u_hbm, i_hbm, ub_hbm, ib_hbm, out_hbm,
             uid_v, iid_v, jju_v, jji_v, ub_v, ib_v, ubuf, ibuf, out_v,
             sem):
    wid = lax.axis_index("s") * 2 + lax.axis_index("c")

    jrow = wid * NJ
    pltpu.sync_copy(uid_hbm.at[pl.ds(jrow, NJ), :], uid_v)
    pltpu.sync_copy(iid_hbm.at[pl.ds(jrow, NJ), :], iid_v)

    bias = []
    for j in range(NJ):
        dst = pl.ds(j * 128, 128)
        bias.append(pltpu.async_copy(ub_hbm.at[uid_v.at[j]],
                                     ub_v.at[dst], sem))
        bias.append(pltpu.async_copy(ib_hbm.at[iid_v.at[j]],
                                     ib_v.at[dst], sem))
    for c in bias:
        c.wait()

    def mkjj(g, _):
        j = g // 8
        sl = pl.ds((g % 8) * 16, 16)
        fl = pl.ds(g * 16, 16)
        u16 = uid_v[j, sl]
        jju_v[fl] = ((u16 >> 7) << 10) + (u16 & 127)
        i16 = iid_v[j, sl]
        jji_v[fl] = ((i16 >> 7) << 10) + (i16 & 127)
        return 0

    lax.fori_loop(0, NJ * 8, mkjj, 0)

    def drain(n):
        for _ in range(n):
            pltpu.make_async_copy(u_hbm.at[jju_v], ubuf.at[0], sem).wait()

    def ui_fire(d, _):
        ub = (d >> 3) * U_G + (d & 7) * 128
        ib = (d >> 3) * I_G + (d & 7) * 128
        pltpu.async_copy(u_hbm.at[pl.ds(ub, U_LEN)].at[jju_v],
                         ubuf.at[d], sem)
        pltpu.async_copy(i_hbm.at[pl.ds(ib, I_LEN)].at[jji_v],
                         ibuf.at[d], sem)

        @pl.when(d >= 16)
        def _():
            drain(2)
        return 0

    lax.fori_loop(0, EMB, ui_fire, 0)
    drain(32)

    def ui_group(g, _):
        sl = pl.ds(g * 16, 16)
        acc = ub_v[sl] + ib_v[sl]
        for d in range(EMB):
            acc = acc + ubuf[d, sl] * ibuf[d, sl]
        out_v[sl] = acc
        return 0

    lax.fori_loop(0, NW, ui_group, 0)

    pltpu.sync_copy(out_v, out_hbm.at[pl.ds(wid * CHUNK, CHUNK)])


def _fields_body(f0, nf, iid_hbm, xf_hbm, fi_hbm, fu_hbm, out_hbm,
                 iid_v, jji_v, xf_v, ftab_v, fib_v, out_v, sem, fsem):
    wid = lax.axis_index("s") * 2 + lax.axis_index("c")
    jrow = wid * NJ
    pltpu.sync_copy(iid_hbm.at[pl.ds(jrow, NJ), :], iid_v)

    def mkjj(g, _):
        j = g // 8
        sl = pl.ds((g % 8) * 16, 16)
        i16 = iid_v[j, sl]
        jji_v[pl.ds(g * 16, 16)] = ((i16 >> 7) << 10) + (i16 & 127)
        return 0

    lax.fori_loop(0, NJ * 8, mkjj, 0)

    def zero(g, _):
        out_v[pl.ds(g * 16, 16)] = jnp.zeros((16,), jnp.float32)
        return 0

    lax.fori_loop(0, NW, zero, 0)

    def drain(n):
        for _ in range(n):
            pltpu.make_async_copy(fu_hbm.at[pl.ds(0, CHUNK)],
                                  fib_v.at[0, 0], sem).wait()

    def fire_field(floc, p):
        def fd(d, _):
            base = (floc * 4 + (d >> 3)) * I_G + (d & 7) * 128
            pltpu.async_copy(fi_hbm.at[pl.ds(base, I_LEN)].at[jji_v],
                             fib_v.at[p, d], sem)
            return 0

        lax.fori_loop(0, FEAT, fd, 0)

    fire_field(0, 0)

    def field(floc, _):
        p = lax.rem(floc, 2)
        fg = floc + f0
        ft = pltpu.async_copy(fu_hbm.at[pl.ds(fg * FU_FLD, FU_FLD)], ftab_v,
                              fsem)
        xc = pltpu.async_copy(xf_hbm.at[fg, pl.ds(jrow, NJ), :], xf_v, fsem)

        @pl.when(floc < nf - 1)
        def _():
            fire_field(floc + 1, 1 - p)

        drain(FEAT)
        ft.wait()
        xc.wait()

        def fgroup(g, _):
            j = g // 8
            sl16 = pl.ds((g % 8) * 16, 16)
            x16 = xf_v[j, sl16]
            xj = ((x16 >> 7) << 10) + (x16 & 127)
            sl = pl.ds(g * 16, 16)
            acc = out_v[sl]
            for d in range(FEAT):
                idx = xj + ((d >> 3) * 8192 + (d & 7) * 128)
                fuv = plsc.load_gather(ftab_v, [idx])
                acc = acc + fuv * fib_v[p, d, sl]
            out_v[sl] = acc
            return 0

        lax.fori_loop(0, NW, fgroup, 0)
        return 0

    lax.fori_loop(0, nf, field, 0)

    pltpu.sync_copy(out_v, out_hbm.at[pl.ds(wid * CHUNK, CHUNK)])


def _sc_ui(mesh, u2, i2, u_t, i_t, ubf, ibf):
    run = pl.kernel(
        _ui_body,
        out_type=jax.ShapeDtypeStruct((B,), jnp.float32),
        mesh=mesh,
        compiler_params=pltpu.CompilerParams(
            needs_layout_passes=False, use_tc_tiling_on_sc=False),
        scratch_types=[
            pltpu.VMEM((NJ, 128), jnp.int32),
            pltpu.VMEM((NJ, 128), jnp.int32),
            pltpu.VMEM((CHUNK,), jnp.int32),
            pltpu.VMEM((CHUNK,), jnp.int32),
            pltpu.VMEM((CHUNK,), jnp.float32),
            pltpu.VMEM((CHUNK,), jnp.float32),
            pltpu.VMEM((EMB, CHUNK), jnp.float32),
            pltpu.VMEM((EMB, CHUNK), jnp.float32),
            pltpu.VMEM((CHUNK,), jnp.float32),
            pltpu.SemaphoreType.DMA,
        ],
    )
    return run(u2, i2, u_t, i_t, ubf, ibf)


def _sc_fields(mesh, f0, nf, i2, xf3, fi_t, fu_t):
    run = pl.kernel(
        functools.partial(_fields_body, f0, nf),
        out_type=jax.ShapeDtypeStruct((B,), jnp.float32),
        mesh=mesh,
        compiler_params=pltpu.CompilerParams(
            needs_layout_passes=False, use_tc_tiling_on_sc=False),
        scratch_types=[
            pltpu.VMEM((NJ, 128), jnp.int32),
            pltpu.VMEM((CHUNK,), jnp.int32),
            pltpu.VMEM((NJ, 128), jnp.int32),
            pltpu.VMEM((FU_FLD,), jnp.float32),
            pltpu.VMEM((2, FEAT, CHUNK), jnp.float32),
            pltpu.VMEM((CHUNK,), jnp.float32),
            pltpu.SemaphoreType.DMA,
            pltpu.SemaphoreType.DMA,
        ],
    )
    return run(i2, xf3, fi_t, fu_t)


def kernel(u_id, i_id, features, user_emb, user_bias, item_emb, item_bias,
           feat_u, feat_i, mean):
    uT = user_emb.T
    iT = item_emb.T
    fuT = jnp.transpose(feat_u, (0, 2, 1)).reshape(NF * FEAT, 1000)
    fiT = jnp.transpose(feat_i, (0, 2, 1)).reshape(NF * FEAT, 100000)

    i_t = _untile(iT, 782, 391).reshape(-1)
    u_t = _untile(uT, 7813, 512).reshape(-1)
    ubf = user_bias.reshape(-1)
    ibf = item_bias.reshape(-1)

    u2 = u_id.astype(jnp.int32).reshape(B // 128, 128)
    i2 = i_id.astype(jnp.int32).reshape(B // 128, 128)
    xf3 = features.astype(jnp.int32).T.reshape(NF, B // 128, 128)

    mesh = plsc.VectorSubcoreMesh(core_axis_name="c", subcore_axis_name="s")
    out_ui = _sc_ui(mesh, u2, i2, u_t, i_t, ubf, ibf)

    fiT, fuT = lax.optimization_barrier((fiT, fuT, u_t, i_t, ubf, ibf))[:2]
    fu_t = _untile(fuT, 8, 8).reshape(-1)
    fi_a = _untile(fiT, 782, 391, 0, NFH * 4).reshape(-1)
    out_fa = _sc_fields(mesh, 0, NFH, i2, xf3, fi_a, fu_t)
    fiT2 = lax.optimization_barrier((fiT, fi_a))[0]
    fi_b = _untile(fiT2, 782, 391, NFH * 4, (NF - NFH) * 4).reshape(-1)
    out_fb = _sc_fields(mesh, NFH, NF - NFH, i2, xf3, fi_b, fu_t)
    return out_ui + out_fa + out_fb + mean[0]
